# baseline (speedup 1.0000x reference)
import functools

import jax
import jax.numpy as jnp
from jax import lax
from jax.experimental import pallas as pl
from jax.experimental.pallas import tpu as pltpu, tpu_sc as plsc

D = 128
NW = 32
CH = 128
WPB = 2
NBUF = 2


def _build(tot):
    per_w = tot // NW
    nch = per_w // CH
    nsb = nch // WPB
    nsteps = nsb // NBUF
    nslot = NBUF * WPB
    mesh = plsc.VectorSubcoreMesh(core_axis_name="c", subcore_axis_name="s")

    @functools.partial(
        pl.kernel,
        mesh=mesh,
        out_type=jax.ShapeDtypeStruct((NW * nsb, WPB * CH, D), jnp.float32),
        scratch_types=[
            pltpu.VMEM((nch, CH), jnp.int32),
            pltpu.VMEM((nslot * CH, D), jnp.float32),
        ]
        + [pltpu.SemaphoreType.DMA for _ in range(nslot + NBUF)],
    )
    def emb(idx_hbm, table_hbm, out_hbm, idx_v, arena, *sems):
        gsem = sems[:nslot]
        wsem = sems[nslot:]
        wid = lax.axis_index("s") * 2 + lax.axis_index("c")
        base = wid * nsb
        pltpu.sync_copy(idx_hbm.at[wid], idx_v)

        def slot(s):
            return arena.at[pl.ds(s * CH, CH)]

        def wslice(p):
            return arena.at[pl.ds(p * WPB * CH, WPB * CH)]

        def gathers(p, grp):
            for q in range(WPB):
                s = p * WPB + q
                pltpu.async_copy(
                    table_hbm.at[idx_v.at[grp * WPB + q]], slot(s), gsem[s]
                )

        def wait_gathers(p, grp):
            for q in range(WPB):
                s = p * WPB + q
                pltpu.make_async_copy(
                    table_hbm.at[idx_v.at[grp * WPB + q]], slot(s), gsem[s]
                ).wait()

        for p in range(NBUF):
            gathers(p, p)

        def body(jo, carry):
            j0 = jo * NBUF
            for p in range(NBUF):
                wait_gathers(p, j0 + p)
                pltpu.async_copy(wslice(p), out_hbm.at[base + j0 + p], wsem[p])
            for p in range(NBUF):
                pltpu.make_async_copy(
                    wslice(p), out_hbm.at[base + j0 + p], wsem[p]
                ).wait()
                gathers(p, j0 + NBUF + p)
            return carry

        lax.fori_loop(0, nsteps - 1, body, 0)

        j0 = (nsteps - 1) * NBUF
        for p in range(NBUF):
            wait_gathers(p, j0 + p)
            pltpu.async_copy(wslice(p), out_hbm.at[base + j0 + p], wsem[p])
        for p in range(NBUF):
            pltpu.make_async_copy(
                wslice(p), out_hbm.at[base + j0 + p], wsem[p]
            ).wait()

    return emb


def kernel(inputs, table):
    b, h = inputs.shape
    tot = b * h
    idx = jnp.asarray(inputs, jnp.int32).reshape(NW, tot // (NW * CH), CH)
    out = _build(tot)(idx, table)
    return out.reshape(b, h, D)


# speedup vs baseline: 1.0134x; 1.0134x over previous
import functools

import jax
import jax.numpy as jnp
from jax import lax
from jax.experimental import pallas as pl
from jax.experimental.pallas import tpu as pltpu, tpu_sc as plsc

D = 128
NW = 32
CH = 128
NBUF = 4


def _build(tot):
    per_w = tot // NW
    nch = per_w // CH
    nsteps = nch // NBUF
    mesh = plsc.VectorSubcoreMesh(core_axis_name="c", subcore_axis_name="s")

    @functools.partial(
        pl.kernel,
        mesh=mesh,
        out_type=jax.ShapeDtypeStruct((tot, D), jnp.float32),
        scratch_types=[
            pltpu.VMEM((nch, CH), jnp.int32),
        ]
        + [pltpu.VMEM((CH, D), jnp.float32) for _ in range(NBUF)]
        + [pltpu.SemaphoreType.DMA for _ in range(2 * NBUF)],
    )
    def emb(idx_hbm, table_hbm, out_hbm, idx_v, *bufs_sems):
        bufs = bufs_sems[:NBUF]
        gsem = bufs_sems[NBUF : 2 * NBUF]
        wsem = bufs_sems[2 * NBUF :]
        wid = lax.axis_index("s") * 2 + lax.axis_index("c")
        base = wid * per_w
        pltpu.sync_copy(idx_hbm.at[wid], idx_v)

        for b in range(NBUF):
            pltpu.async_copy(table_hbm.at[idx_v.at[b]], bufs[b], gsem[b])

        def body(jo, carry):
            j0 = jo * NBUF
            for b in range(NBUF):
                pltpu.make_async_copy(
                    table_hbm.at[idx_v.at[j0 + b]], bufs[b], gsem[b]
                ).wait()
                pltpu.async_copy(
                    bufs[b], out_hbm.at[pl.ds(base + (j0 + b) * CH, CH)], wsem[b]
                )
            jn = j0 + NBUF
            for b in range(NBUF):
                pltpu.make_async_copy(
                    bufs[b], out_hbm.at[pl.ds(base + (j0 + b) * CH, CH)], wsem[b]
                ).wait()
                pltpu.async_copy(table_hbm.at[idx_v.at[jn + b]], bufs[b], gsem[b])
            return carry

        lax.fori_loop(0, nsteps - 1, body, 0)

        j0 = (nsteps - 1) * NBUF
        for b in range(NBUF):
            pltpu.make_async_copy(
                table_hbm.at[idx_v.at[j0 + b]], bufs[b], gsem[b]
            ).wait()
            pltpu.async_copy(
                bufs[b], out_hbm.at[pl.ds(base + (j0 + b) * CH, CH)], wsem[b]
            )
        for b in range(NBUF):
            pltpu.make_async_copy(
                bufs[b], out_hbm.at[pl.ds(base + (j0 + b) * CH, CH)], wsem[b]
            ).wait()

    return emb


def kernel(inputs, table):
    b, h = inputs.shape
    tot = b * h
    idx = jnp.asarray(inputs, jnp.int32).reshape(NW, tot // (NW * CH), CH)
    out = _build(tot)(idx, table)
    return out.reshape(b, h, D)


# speedup vs baseline: 1.0173x; 1.0038x over previous
import functools

import jax
import jax.numpy as jnp
from jax import lax
from jax.experimental import pallas as pl
from jax.experimental.pallas import tpu as pltpu, tpu_sc as plsc

D = 128
NW = 32
CH = 128
NBUF = 6


def _build(tot):
    per_w = tot // NW
    nch = per_w // CH
    nfull = nch // NBUF
    rem = nch % NBUF
    mesh = plsc.VectorSubcoreMesh(core_axis_name="c", subcore_axis_name="s")

    @functools.partial(
        pl.kernel,
        mesh=mesh,
        out_type=jax.ShapeDtypeStruct((tot, D), jnp.float32),
        scratch_types=[
            pltpu.VMEM((nch, CH), jnp.int32),
        ]
        + [pltpu.VMEM((CH, D), jnp.float32) for _ in range(NBUF)]
        + [pltpu.SemaphoreType.DMA for _ in range(2 * NBUF)],
    )
    def emb(idx_hbm, table_hbm, out_hbm, idx_v, *bufs_sems):
        bufs = bufs_sems[:NBUF]
        gsem = bufs_sems[NBUF : 2 * NBUF]
        wsem = bufs_sems[2 * NBUF :]
        wid = lax.axis_index("s") * 2 + lax.axis_index("c")
        base = wid * per_w
        pltpu.sync_copy(idx_hbm.at[wid], idx_v)

        for b in range(NBUF):
            pltpu.async_copy(table_hbm.at[idx_v.at[b]], bufs[b], gsem[b])

        def body(jo, carry):
            j0 = jo * NBUF
            for b in range(NBUF):
                pltpu.make_async_copy(
                    table_hbm.at[idx_v.at[j0 + b]], bufs[b], gsem[b]
                ).wait()
                pltpu.async_copy(
                    bufs[b], out_hbm.at[pl.ds(base + (j0 + b) * CH, CH)], wsem[b]
                )
            jn = j0 + NBUF
            for b in range(NBUF):
                pltpu.make_async_copy(
                    bufs[b], out_hbm.at[pl.ds(base + (j0 + b) * CH, CH)], wsem[b]
                ).wait()
                pltpu.async_copy(table_hbm.at[idx_v.at[jn + b]], bufs[b], gsem[b])
            return carry

        lax.fori_loop(0, nfull - 1, body, 0)

        j0 = (nfull - 1) * NBUF
        for b in range(NBUF):
            pltpu.make_async_copy(
                table_hbm.at[idx_v.at[j0 + b]], bufs[b], gsem[b]
            ).wait()
            pltpu.async_copy(
                bufs[b], out_hbm.at[pl.ds(base + (j0 + b) * CH, CH)], wsem[b]
            )
        for b in range(NBUF):
            pltpu.make_async_copy(
                bufs[b], out_hbm.at[pl.ds(base + (j0 + b) * CH, CH)], wsem[b]
            ).wait()
            if b < rem:
                pltpu.async_copy(
                    table_hbm.at[idx_v.at[j0 + NBUF + b]], bufs[b], gsem[b]
                )
        j1 = j0 + NBUF
        for b in range(rem):
            pltpu.make_async_copy(
                table_hbm.at[idx_v.at[j1 + b]], bufs[b], gsem[b]
            ).wait()
            pltpu.async_copy(
                bufs[b], out_hbm.at[pl.ds(base + (j1 + b) * CH, CH)], wsem[b]
            )
        for b in range(rem):
            pltpu.make_async_copy(
                bufs[b], out_hbm.at[pl.ds(base + (j1 + b) * CH, CH)], wsem[b]
            ).wait()

    return emb


def kernel(inputs, table):
    b, h = inputs.shape
    tot = b * h
    idx = jnp.asarray(inputs, jnp.int32).reshape(NW, tot // (NW * CH), CH)
    out = _build(tot)(idx, table)
    return out.reshape(b, h, D)
